# Initial kernel scaffold; baseline (speedup 1.0000x reference)
import jax
import jax.numpy as jnp
from jax.experimental import pallas as pl


def kernel(x, edge_index, edge_attr, glob, batch, W_edge, b_edge, W_le, b_le, W_n, b_n, W_n2, b_n2, W_g, b_g, a):
    raise NotImplementedError("write your pallas kernel here")


# speedup vs baseline: 12.5845x; 12.5845x over previous
import functools

import jax
import jax.numpy as jnp
from jax import lax
from jax.experimental import pallas as pl
from jax.experimental.pallas import tpu as pltpu
from jax.experimental.pallas import tpu_sc as plsc

N = 10000
E = 320000
G = 16
D = 128
DE = 16

_HI = jax.lax.Precision.HIGHEST


_NB1 = 400


def _tc1_body(x_ref, we1_ref, m_ref, c1_ref, q_ref, p_ref):
    xb = x_ref[...]
    q_ref[...] = jnp.dot(xb, we1_ref[...], preferred_element_type=jnp.float32,
                         precision=_HI)
    p_ref[...] = jnp.dot(xb, m_ref[...], preferred_element_type=jnp.float32,
                         precision=_HI) + c1_ref[...]


def _tc1(x, we1, m, c1):
    nblk = N // _NB1
    return pl.pallas_call(
        _tc1_body,
        grid=(nblk,),
        in_specs=[
            pl.BlockSpec((_NB1, D), lambda i: (i, 0)),
            pl.BlockSpec((D, DE), lambda i: (0, 0)),
            pl.BlockSpec((D, D), lambda i: (0, 0)),
            pl.BlockSpec((1, D), lambda i: (0, 0)),
        ],
        out_specs=[
            pl.BlockSpec((_NB1, DE), lambda i: (i, 0)),
            pl.BlockSpec((_NB1, D), lambda i: (i, 0)),
        ],
        out_shape=[
            jax.ShapeDtypeStruct((N, DE), jnp.float32),
            jax.ShapeDtypeStruct((N, D), jnp.float32),
        ],
    )(x, we1, m, c1)


_NC = 2
_NS = 16
_NW = _NC * _NS
_EPW = E // _NW
_CH = 2000
_NCHUNK = _EPW // _CH


def _sc_body(q_hbm, row_hbm, col_hbm, g_hbm, maskp_hbm,
             idx_v, rows_v, mask_v, sem):
    wid = lax.axis_index("s") * _NC + lax.axis_index("c")
    base = wid * _EPW

    def _zero(j, carry):
        mask_v[pl.ds(j * 16, 16)] = jnp.zeros((16,), jnp.float32)
        return carry

    lax.fori_loop(0, N // 16, _zero, 0)

    ones16 = jnp.ones((16,), jnp.float32)
    for c in range(_NCHUNK):
        cb = base + c * _CH
        pltpu.sync_copy(row_hbm.at[pl.ds(cb, _CH)], idx_v)
        pltpu.async_copy(q_hbm.at[idx_v], rows_v, sem).wait()
        pltpu.sync_copy(rows_v, g_hbm.at[pl.ds(cb, _CH)])
        pltpu.sync_copy(col_hbm.at[pl.ds(cb, _CH)], idx_v)

        def _scat(j, carry):
            iv = idx_v[pl.ds(j * 16, 16)]
            plsc.store_scatter(mask_v, [iv], ones16)
            return carry

        lax.fori_loop(0, _CH // 16, _scat, 0)

    pltpu.sync_copy(mask_v, maskp_hbm.at[wid])


def _sc_gather_mask(q, row, col):
    mesh = plsc.VectorSubcoreMesh(core_axis_name="c", subcore_axis_name="s",
                                  num_cores=_NC, num_subcores=_NS)
    k = functools.partial(
        pl.kernel,
        out_type=[
            jax.ShapeDtypeStruct((E, DE), jnp.float32),
            jax.ShapeDtypeStruct((_NW, N), jnp.float32),
        ],
        mesh=mesh,
        scratch_types=[
            pltpu.VMEM((_CH,), jnp.int32),
            pltpu.VMEM((_CH, DE), jnp.float32),
            pltpu.VMEM((N,), jnp.float32),
            pltpu.SemaphoreType.DMA,
        ],
        compiler_params=pltpu.CompilerParams(needs_layout_passes=False,
                                             use_tc_tiling_on_sc=False),
    )(_sc_body)
    return k(q, row, col)


_EB = 2000


def _tc2_body(g_ref, ea_ref, w2_ref, be_ref, e_ref):
    acc = jnp.dot(ea_ref[...], w2_ref[...], preferred_element_type=jnp.float32,
                  precision=_HI)
    e_ref[...] = jnp.maximum(g_ref[...] + acc + be_ref[...], 0.0)


def _tc2(g, ea, w2, be):
    nblk = E // _EB
    return pl.pallas_call(
        _tc2_body,
        grid=(nblk,),
        in_specs=[
            pl.BlockSpec((_EB, DE), lambda i: (i, 0)),
            pl.BlockSpec((_EB, DE), lambda i: (i, 0)),
            pl.BlockSpec((DE, DE), lambda i: (0, 0)),
            pl.BlockSpec((1, DE), lambda i: (0, 0)),
        ],
        out_specs=pl.BlockSpec((_EB, DE), lambda i: (i, 0)),
        out_shape=jax.ShapeDtypeStruct((E, DE), jnp.float32),
    )(g, ea, w2, be)


_NB3 = 400


def _tc3_body(p_ref, maskp_ref, batch_ref, glob_ref, wn2b_ref, bn2_ref,
              wg1_ref, wg2_ref, bg_ref, xn_ref, un_ref, acc_ref, cnt_ref):
    i = pl.program_id(0)
    nblk = pl.num_programs(0)

    @pl.when(i == 0)
    def _():
        acc_ref[...] = jnp.zeros_like(acc_ref)
        cnt_ref[...] = jnp.zeros_like(cnt_ref)

    m = (jnp.sum(maskp_ref[...], axis=1, keepdims=True) > 0.0
         ).astype(jnp.float32)
    oh = (batch_ref[...] ==
          lax.broadcasted_iota(jnp.int32, (_NB3, G), 1)).astype(jnp.float32)
    gb = jnp.dot(glob_ref[...], wn2b_ref[...],
                 preferred_element_type=jnp.float32, precision=_HI)
    xn = jnp.maximum(
        p_ref[...] * m
        + jnp.dot(oh, gb, preferred_element_type=jnp.float32, precision=_HI)
        + bn2_ref[...], 0.0)
    xn_ref[...] = xn
    acc_ref[...] += lax.dot_general(oh, xn, (((0,), (0,)), ((), ())),
                                    preferred_element_type=jnp.float32,
                                    precision=_HI)
    cnt_ref[...] += lax.dot_general(oh, jnp.ones((_NB3, D), jnp.float32),
                                    (((0,), (0,)), ((), ())),
                                    preferred_element_type=jnp.float32,
                                    precision=_HI)

    @pl.when(i == nblk - 1)
    def _():
        mean = acc_ref[...] / jnp.maximum(cnt_ref[...], 1.0)
        u = jnp.dot(glob_ref[...], wg1_ref[...],
                    preferred_element_type=jnp.float32, precision=_HI)
        u += jnp.dot(mean, wg2_ref[...],
                     preferred_element_type=jnp.float32, precision=_HI)
        un_ref[...] = jnp.maximum(u + bg_ref[...], 0.0)


def _tc3(p, maskp, batch2, glob, wn2b, bn2, wg1, wg2, bg):
    nblk = N // _NB3
    return pl.pallas_call(
        _tc3_body,
        grid=(nblk,),
        in_specs=[
            pl.BlockSpec((_NB3, D), lambda i: (i, 0)),
            pl.BlockSpec((_NB3, _NW), lambda i: (i, 0)),
            pl.BlockSpec((_NB3, 1), lambda i: (i, 0)),
            pl.BlockSpec((G, D), lambda i: (0, 0)),
            pl.BlockSpec((D, D), lambda i: (0, 0)),
            pl.BlockSpec((1, D), lambda i: (0, 0)),
            pl.BlockSpec((D, D), lambda i: (0, 0)),
            pl.BlockSpec((D, D), lambda i: (0, 0)),
            pl.BlockSpec((1, D), lambda i: (0, 0)),
        ],
        out_specs=[
            pl.BlockSpec((_NB3, D), lambda i: (i, 0)),
            pl.BlockSpec((G, D), lambda i: (0, 0)),
        ],
        out_shape=[
            jax.ShapeDtypeStruct((N, D), jnp.float32),
            jax.ShapeDtypeStruct((G, D), jnp.float32),
        ],
        scratch_shapes=[
            pltpu.VMEM((G, D), jnp.float32),
            pltpu.VMEM((G, D), jnp.float32),
        ],
    )(p, maskp, batch2, glob, wn2b, bn2, wg1, wg2, bg)


def kernel(x, edge_index, edge_attr, glob, batch, W_edge, b_edge, W_le, b_le,
           W_n, b_n, W_n2, b_n2, W_g, b_g, a):
    del W_le, b_le, a
    row = edge_index[0]
    col = edge_index[1]

    we1 = W_edge[:D]
    w2 = W_edge[D:]
    wn2a = W_n2[:D]
    wn2b = W_n2[D:]
    m_fold = jnp.dot(W_n, wn2a, precision=_HI)
    c1 = (b_n @ wn2a)[None, :]

    q, p = _tc1(x, we1, m_fold, c1)
    g_rows, maskp = _sc_gather_mask(q, row, col)
    e = _tc2(g_rows, edge_attr, w2, b_edge[None, :])
    x_new, u_new = _tc3(p, maskp.T, batch[:, None], glob, wn2b, b_n2[None, :],
                        W_g[:D], W_g[D:], b_g[None, :])
    return (x_new, e, u_new)
